# baseline (speedup 1.0000x reference)
import functools
import math

import jax
import jax.numpy as jnp
import numpy as np
from jax.experimental import pallas as pl
from jax.experimental.pallas import tpu as pltpu

_L = 2048
_C = 2048
_E = 64
_K = 32
_BLK = 256

_I = False


def _pos_embedding(L, C):
    position = jnp.arange(L, dtype=jnp.float32)[:, None]
    div_term = jnp.exp(jnp.arange(0, C, 2, dtype=jnp.float32) * (-math.log(10000.0) / C))
    pe = jnp.zeros((L, C), dtype=jnp.float32)
    pe = pe.at[:, 0::2].set(jnp.sin(position * div_term))
    pe = pe.at[:, 1::2].set(jnp.cos(position * div_term))
    return pe


_POS = np.asarray(_pos_embedding(_L, _C))


def _logits_body(pos_ref, w1_ref, b1_ref, w2_ref, out_ref):
    h = jnp.dot(pos_ref[...], w1_ref[...], preferred_element_type=jnp.float32)
    h = h + b1_ref[...]
    h = 0.5 * h * (1.0 + jax.lax.erf(h * (1.0 / math.sqrt(2.0))))
    out_ref[...] = jnp.dot(h, w2_ref[...], preferred_element_type=jnp.float32)


def _topk_body(l_ref, g_ref, i_ref, *, batch):
    l = jnp.transpose(l_ref[...])
    m = jnp.max(l, axis=1, keepdims=True)
    e = jnp.exp(l - m)
    p = e / jnp.sum(e, axis=1, keepdims=True)

    iota_l = jax.lax.broadcasted_iota(jnp.int32, (_E, _L), 1)
    iota_k = jax.lax.broadcasted_iota(jnp.int32, (_E, _K), 1)

    def body(s, carry):
        a, idxs = carry
        mx = jnp.max(a, axis=1, keepdims=True)
        hit = a == mx
        idx = jnp.min(jnp.where(hit, iota_l, _L), axis=1, keepdims=True)
        sel = iota_l == idx
        a = jnp.where(sel, -1.0, a)
        idxs = jnp.where(iota_k == s, idx, idxs)
        return a, idxs

    a = p
    idxs = jnp.zeros((_E, _K), jnp.int32)
    for s in range(_K):
        a, idxs = body(s, (a, idxs))
    g = jnp.where(a < 0.0, p, 0.0)
    g_ref[...] = jnp.broadcast_to(g[None], (batch, _E, _L))
    i_ref[...] = jnp.broadcast_to(idxs[None], (batch, _E, _K))


def kernel(input_features, W1, b1, W2):
    B, L, C = input_features.shape
    pos = jnp.asarray(_POS)

    logits = pl.pallas_call(
        _logits_body,
        grid=(L // _BLK,),
        in_specs=[
            pl.BlockSpec((_BLK, C), lambda i: (i, 0)),
            pl.BlockSpec((C, C), lambda i: (0, 0)),
            pl.BlockSpec((1, C), lambda i: (0, 0)),
            pl.BlockSpec((C, _E), lambda i: (0, 0)),
        ],
        out_specs=pl.BlockSpec((_BLK, _E), lambda i: (i, 0)),
        out_shape=jax.ShapeDtypeStruct((L, _E), jnp.float32),
        interpret=_I,
    )(pos, W1, b1.reshape(1, C), W2)

    G, I = pl.pallas_call(
        functools.partial(_topk_body, batch=B),
        out_shape=(
            jax.ShapeDtypeStruct((B, _E, L), jnp.float32),
            jax.ShapeDtypeStruct((B, _E, _K), jnp.int32),
        ),
        interpret=_I,
    )(logits)

    return (G, I)


# speedup vs baseline: 1.0443x; 1.0443x over previous
import functools
import math

import jax
import jax.numpy as jnp
import numpy as np
from jax.experimental import pallas as pl
from jax.experimental.pallas import tpu as pltpu

_L = 2048
_C = 2048
_E = 64
_K = 32
_BLK = 256

_I = False


def _pos_embedding(L, C):
    position = jnp.arange(L, dtype=jnp.float32)[:, None]
    div_term = jnp.exp(jnp.arange(0, C, 2, dtype=jnp.float32) * (-math.log(10000.0) / C))
    pe = jnp.zeros((L, C), dtype=jnp.float32)
    pe = pe.at[:, 0::2].set(jnp.sin(position * div_term))
    pe = pe.at[:, 1::2].set(jnp.cos(position * div_term))
    return pe


_POS = np.asarray(_pos_embedding(_L, _C))


def _fused_body(pos_ref, w1_ref, b1_ref, w2_ref, g_ref, i_ref, l_scr, *, batch):
    step = pl.program_id(0)
    h = jnp.dot(pos_ref[...], w1_ref[...], preferred_element_type=jnp.float32)
    h = h + b1_ref[...]
    h = 0.5 * h * (1.0 + jax.lax.erf(h * (1.0 / math.sqrt(2.0))))
    l_scr[pl.ds(step * _BLK, _BLK), :] = jnp.dot(
        h, w2_ref[...], preferred_element_type=jnp.float32)

    @pl.when(step == _L // _BLK - 1)
    def _topk():
        l = jnp.transpose(l_scr[...])
        m = jnp.max(l, axis=1, keepdims=True)
        e = jnp.exp(l - m)
        p = e / jnp.sum(e, axis=1, keepdims=True)

        iota_l = jax.lax.broadcasted_iota(jnp.int32, (_E, _L), 1)
        iota_k = jax.lax.broadcasted_iota(jnp.int32, (_E, _K), 1)

        a = p
        idxs = jnp.zeros((_E, _K), jnp.int32)
        for s in range(_K):
            mx = jnp.max(a, axis=1, keepdims=True)
            hit = a == mx
            idx = jnp.min(jnp.where(hit, iota_l, _L), axis=1, keepdims=True)
            sel = iota_l == idx
            a = jnp.where(sel, -1.0, a)
            idxs = jnp.where(iota_k == s, idx, idxs)

        g = jnp.where(a < 0.0, p, 0.0)
        g_ref[...] = jnp.broadcast_to(g[None], (batch, _E, _L))
        i_ref[...] = jnp.broadcast_to(idxs[None], (batch, _E, _K))


def kernel(input_features, W1, b1, W2):
    B, L, C = input_features.shape
    pos = jnp.asarray(_POS)

    G, I = pl.pallas_call(
        functools.partial(_fused_body, batch=B),
        grid=(L // _BLK,),
        in_specs=[
            pl.BlockSpec((_BLK, C), lambda i: (i, 0)),
            pl.BlockSpec((C, C), lambda i: (0, 0)),
            pl.BlockSpec((1, C), lambda i: (0, 0)),
            pl.BlockSpec((C, _E), lambda i: (0, 0)),
        ],
        out_specs=(
            pl.BlockSpec((B, _E, L), lambda i: (0, 0, 0)),
            pl.BlockSpec((B, _E, _K), lambda i: (0, 0, 0)),
        ),
        out_shape=(
            jax.ShapeDtypeStruct((B, _E, L), jnp.float32),
            jax.ShapeDtypeStruct((B, _E, _K), jnp.int32),
        ),
        scratch_shapes=[pltpu.VMEM((L, _E), jnp.float32)],
        interpret=_I,
    )(pos, W1, b1.reshape(1, C), W2)

    return (G, I)
